# Initial kernel scaffold; baseline (speedup 1.0000x reference)
import jax
import jax.numpy as jnp
from jax.experimental import pallas as pl


def kernel(x, rbf, sbf, idx_kj, idx_ji, W_rbf1, W_rbf2, W_sbf1, W_sbf2, W_rbf, W_kj, b_kj, W_ji, b_ji, W_down, W_up, W_lin, b_lin, res_W1, res_b1, res_W2, res_b2):
    raise NotImplementedError("write your pallas kernel here")


# speedup vs baseline: 7.9062x; 7.9062x over previous
import dataclasses
import functools

import jax
import jax.numpy as jnp
from jax import lax
from jax.experimental import pallas as pl
from jax.experimental.pallas import tpu as pltpu
from jax.experimental.pallas import tpu_sc as plsc

NC = 2
NS = 16
CHUNK = 7936
NBIN = 21
CAP = 1280
ACC_ROWS = 8064

_SC_CP = pltpu.CompilerParams()
if "needs_layout_passes" in pltpu.CompilerParams.__dataclass_fields__:
    _SC_CP = dataclasses.replace(_SC_CP, needs_layout_passes=False)


def _swish(v):
    return v * (0.5 * jnp.tanh(v * 0.5) + 0.5)


def _swish_bf(v):
    vb = v.astype(jnp.bfloat16)
    half = jnp.bfloat16(0.5)
    return vb * (half * jnp.tanh(vb * half) + half)


def _k1_body(x_ref, rbf_ref, wji, bji, wkj, bkj, wr1, wr2, wdown,
             xji_out, xd_out):
    xb = x_ref[...].astype(jnp.bfloat16)
    xji = jnp.dot(xb, wji[...], preferred_element_type=jnp.float32) + bji[...]
    xji_out[...] = _swish_bf(xji)
    xkj = _swish_bf(jnp.dot(xb, wkj[...], preferred_element_type=jnp.float32)
                    + bkj[...]).astype(jnp.float32)
    rbf_e = jnp.dot(jnp.dot(rbf_ref[...], wr1[...],
                            preferred_element_type=jnp.float32),
                    wr2[...], preferred_element_type=jnp.float32)
    xkj = xkj * rbf_e
    xd = jnp.dot(xkj.astype(jnp.bfloat16), wdown[...],
                 preferred_element_type=jnp.float32)
    xd_out[:, :xd.shape[1]] = _swish(xd)


def _pre(x, rbf, wji, bji, wkj, bkj, wr1, wr2, wdown):
    e, h = x.shape
    nr = rbf.shape[1]
    nint = wdown.shape[1]
    be = 3200
    full = lambda i: (0, 0)
    row = lambda i: (i, 0)
    return pl.pallas_call(
        _k1_body,
        grid=(e // be,),
        in_specs=[
            pl.BlockSpec((be, h), row),
            pl.BlockSpec((be, nr), row),
            pl.BlockSpec((h, h), full),
            pl.BlockSpec((1, h), full),
            pl.BlockSpec((h, h), full),
            pl.BlockSpec((1, h), full),
            pl.BlockSpec((nr, 16), full),
            pl.BlockSpec((16, h), full),
            pl.BlockSpec((h, nint), full),
        ],
        out_specs=[pl.BlockSpec((be, h), row),
                   pl.BlockSpec((be, 2 * nint), row)],
        out_shape=[jax.ShapeDtypeStruct((e, h), jnp.bfloat16),
                   jax.ShapeDtypeStruct((e, 2 * nint), jnp.float32)],
    )(x, rbf, wji, bji, wkj, bkj, wr1, wr2, wdown)


def _k2_body(sbf_ref, g_ref, w1, w2, v_out):
    s16 = jnp.dot(sbf_ref[...], w1[...], preferred_element_type=jnp.float32)
    se = jnp.dot(s16, w2[...], preferred_element_type=jnp.float32)
    prod = se * g_ref[:, :se.shape[1]]
    v_out[:, :prod.shape[1]] = prod


def _mid(sbf, g, w1, w2):
    t, nsb = sbf.shape
    nint = w2.shape[1]
    bt = 12800
    full = lambda i: (0, 0)
    row = lambda i: (i, 0)
    return pl.pallas_call(
        _k2_body,
        grid=(t // bt,),
        in_specs=[
            pl.BlockSpec((bt, nsb), row),
            pl.BlockSpec((bt, 2 * nint), row),
            pl.BlockSpec((nsb, 16), full),
            pl.BlockSpec((16, nint), full),
        ],
        out_specs=[pl.BlockSpec((bt, 2 * nint), row)],
        out_shape=[jax.ShapeDtypeStruct((t, 2 * nint), jnp.float32)],
    )(sbf, g, w1, w2)[0]


def _k3_body(seg_ref, xji_ref, x_ref, rbf_ref, wup,
             rw10, rw11, rw12, rw20, rw21, rw22,
             rb10, rb11, rb12, rb20, rb21, rb22,
             wlin, blin, wrbf, e1_out, e2_out):
    def res_layer(h, w1, b1, w2, b2):
        u = _swish_bf(jnp.dot(h.astype(jnp.bfloat16), w1[...],
                              preferred_element_type=jnp.float32) + b1[...])
        u = _swish_bf(jnp.dot(u, w2[...],
                              preferred_element_type=jnp.float32) + b2[...])
        return h + u.astype(jnp.float32)

    nint = wup.shape[0]
    xkj = _swish_bf(jnp.dot(seg_ref[:, :nint].astype(jnp.bfloat16), wup[...],
                            preferred_element_type=jnp.float32))
    e1 = xji_ref[...].astype(jnp.float32) + xkj.astype(jnp.float32)
    e1 = res_layer(e1, rw10, rb10, rw20, rb20)
    e1 = _swish_bf(jnp.dot(e1.astype(jnp.bfloat16), wlin[...],
                           preferred_element_type=jnp.float32) + blin[...]
                   ).astype(jnp.float32)
    e1 = e1 + x_ref[...]
    e1 = res_layer(e1, rw11, rb11, rw21, rb21)
    e1 = res_layer(e1, rw12, rb12, rw22, rb22)
    e1_out[...] = e1
    e2_out[...] = jnp.dot(rbf_ref[...], wrbf[...],
                          preferred_element_type=jnp.float32) * e1


def _post(seg, x_ji, x, rbf, wup, rws1, rbs1, rws2, rbs2, wlin, blin, wrbf):
    e, h = x.shape
    nr = rbf.shape[1]
    nint = wup.shape[0]
    be = 3200
    full = lambda i: (0, 0)
    row = lambda i: (i, 0)
    rw1 = [rws1[i] for i in range(3)]
    rw2 = [rws2[i] for i in range(3)]
    rb1 = [rbs1[i:i + 1] for i in range(3)]
    rb2 = [rbs2[i:i + 1] for i in range(3)]
    wspec = [pl.BlockSpec((h, h), full)] * 6
    bspec = [pl.BlockSpec((1, h), full)] * 6
    return pl.pallas_call(
        _k3_body,
        grid=(e // be,),
        in_specs=[
            pl.BlockSpec((be, 2 * nint), row),
            pl.BlockSpec((be, h), row),
            pl.BlockSpec((be, h), row),
            pl.BlockSpec((be, nr), row),
            pl.BlockSpec((nint, h), full),
        ] + wspec + bspec + [
            pl.BlockSpec((h, h), full),
            pl.BlockSpec((1, h), full),
            pl.BlockSpec((nr, h), full),
        ],
        out_specs=[pl.BlockSpec((be, h), row), pl.BlockSpec((be, h), row)],
        out_shape=[jax.ShapeDtypeStruct((e, h), jnp.float32),
                   jax.ShapeDtypeStruct((e, h), jnp.float32)],
    )(seg, x_ji, x, rbf, wup, *rw1, *rw2, *rb1, *rb2, wlin, blin, wrbf)


def _sc_gather(xd, idx):
    e, w128 = xd.shape
    t = idx.shape[0]
    gw = 256
    nstr = gw // 128
    nwin = t // gw
    nw = NC * NS
    outer = (nwin + nw - 1) // nw
    mesh = plsc.VectorSubcoreMesh(core_axis_name="c", subcore_axis_name="s")

    @functools.partial(
        pl.kernel, mesh=mesh,
        out_type=jax.ShapeDtypeStruct((t, w128), jnp.float32),
        scratch_types=[pltpu.VMEM((nstr, 128), jnp.int32),
                       pltpu.VMEM((nstr, 128), jnp.int32),
                       pltpu.VMEM((gw, w128), jnp.float32),
                       pltpu.VMEM((gw, w128), jnp.float32),
                       pltpu.SemaphoreType.DMA,
                       pltpu.SemaphoreType.DMA,
                       pltpu.SemaphoreType.DMA,
                       pltpu.SemaphoreType.DMA,
                       pltpu.SemaphoreType.DMA,
                       pltpu.SemaphoreType.DMA],
    )
    def k(xd_hbm, idx_hbm, g_hbm, idx0, idx1, rows0, rows1,
          gsem0, gsem1, isem0, isem1, wsem0, wsem1):
        cid = lax.axis_index("c")
        sid = lax.axis_index("s")
        wid = sid * NC + cid
        ibufs = (idx0, idx1)
        rbufs = (rows0, rows1)
        isems = (isem0, isem1)
        gsems = (gsem0, gsem1)
        wsems = (wsem0, wsem1)

        def fire_idx(wi, par):
            @pl.when(wi < nwin)
            def _():
                t0 = wi * gw
                for k8 in range(nstr):
                    pltpu.async_copy(idx_hbm.at[pl.ds(t0 + k8 * 128, 128)],
                                     ibufs[par].at[k8], isems[par])

        def do_window(wi, par):
            @pl.when(wi < nwin)
            def _():
                @pl.when(wi >= 2 * nw)
                def _():
                    pltpu.make_async_copy(
                        rbufs[par], g_hbm.at[pl.ds(0, gw)], wsems[par]).wait()

                for k8 in range(nstr):
                    pltpu.make_async_copy(idx_hbm.at[pl.ds(0, 128)],
                                          ibufs[par].at[k8],
                                          isems[par]).wait()
                fire_idx(wi + nw, 1 - par)
                copies = [
                    pltpu.async_copy(xd_hbm.at[ibufs[par].at[k8]],
                                     rbufs[par].at[pl.ds(k8 * 128, 128)],
                                     gsems[par])
                    for k8 in range(nstr)
                ]
                for c in copies:
                    c.wait()
                pltpu.async_copy(rbufs[par], g_hbm.at[pl.ds(wi * gw, gw)],
                                 wsems[par])

        fire_idx(wid, 0)

        @pl.loop(0, (outer + 1) // 2)
        def _(u):
            do_window(u * 2 * nw + wid, 0)
            do_window((u * 2 + 1) * nw + wid, 1)

        for par in range(2):
            pltpu.make_async_copy(rbufs[par], g_hbm.at[pl.ds(0, gw)],
                                  wsems[par]).wait()

    return k(xd, idx)


def _sc_bin(idx, t):
    nw = NC * NS
    win = 256
    nwin = t // win
    outer = (nwin + nw - 1) // nw
    mesh = plsc.VectorSubcoreMesh(core_axis_name="c", subcore_axis_name="s")
    nent = NBIN * nw * CAP

    @functools.partial(
        pl.kernel, mesh=mesh, compiler_params=_SC_CP,
        out_type=[jax.ShapeDtypeStruct((nent,), jnp.int32),
                  jax.ShapeDtypeStruct((nent,), jnp.int32)],
        scratch_types=[pltpu.VMEM((win,), jnp.int32),
                       pltpu.VMEM((NBIN * CAP,), jnp.int32),
                       pltpu.VMEM((NBIN * CAP,), jnp.int32),
                       pltpu.SMEM((NBIN,), jnp.int32),
                       pltpu.SemaphoreType.DMA],
    )
    def k(ji_hbm, bt_hbm, bj_hbm, jiw, tbuf, jbuf, ptrs, sem):
        cid = lax.axis_index("c")
        sid = lax.axis_index("s")
        wid = sid * NC + cid
        lanes = lax.iota(jnp.int32, 16)
        padt = (wid * 16 + lanes) * 64
        padj = CHUNK + wid * 4 + (lanes & 3)

        @pl.loop(0, NBIN * CAP // 16)
        def _(z):
            tbuf[pl.ds(z * 16, 16)] = padt
            jbuf[pl.ds(z * 16, 16)] = padj

        for b in range(NBIN):
            ptrs[b] = 0

        @pl.loop(0, outer)
        def _(w):
            wi = w * nw + wid

            @pl.when(wi < nwin)
            def _():
                t0 = wi * win
                pltpu.sync_copy(ji_hbm.at[pl.ds(t0, win)], jiw)

                @pl.loop(0, win // 16)
                def _(i):
                    ji16 = jiw[pl.ds(i * 16, 16)]
                    binv = ji16 // CHUNK
                    tvec = t0 + i * 16 + lanes
                    for b in range(NBIN):
                        m = binv == b
                        mi = m.astype(jnp.int32)
                        cnt = jnp.sum(mi)

                        @pl.when(cnt > 0)
                        def _():
                            cum = plsc.cumsum(mi)
                            ptr = ptrs[b]
                            pos = b * CAP + ptr + cum - 1
                            plsc.store_scatter(tbuf, [pos], tvec, mask=m)
                            plsc.store_scatter(jbuf, [pos], ji16 - b * CHUNK,
                                               mask=m)
                            ptrs[b] = ptr + cnt

        for b in range(NBIN):
            dst = (b * nw + wid) * CAP
            pltpu.sync_copy(tbuf.at[pl.ds(b * CAP, CAP)],
                            bt_hbm.at[pl.ds(dst, CAP)])
            pltpu.sync_copy(jbuf.at[pl.ds(b * CAP, CAP)],
                            bj_hbm.at[pl.ds(dst, CAP)])

    return k(idx)


def _sc_scatter(v, bins_t, bins_j, e):
    t, w128 = v.shape
    nw = NC * NS
    rpt = CHUNK // NS
    ent_chunk = nw * CAP
    ent_tile = ent_chunk // NS
    win = 256
    nwin = ent_tile // win
    npass = (NBIN + 1) // NC
    rem = e - (NBIN - 1) * CHUNK
    rem_full = rem // rpt
    rem_part = rem - rem_full * rpt
    mesh = plsc.VectorSubcoreMesh(core_axis_name="c", subcore_axis_name="s")

    @functools.partial(
        pl.kernel, mesh=mesh,
        out_type=jax.ShapeDtypeStruct((e, w128), jnp.float32),
        scratch_types=[pltpu.VMEM((2, 128), jnp.int32),
                       pltpu.VMEM((2, 128), jnp.int32),
                       pltpu.VMEM((2, 128), jnp.int32),
                       pltpu.VMEM((2, 128), jnp.int32),
                       pltpu.VMEM((win, w128), jnp.float32),
                       pltpu.VMEM((win, w128), jnp.float32),
                       pltpu.VMEM_SHARED((ACC_ROWS, w128), jnp.float32),
                       pltpu.SemaphoreType.DMA,
                       pltpu.SemaphoreType.DMA,
                       pltpu.SemaphoreType.DMA,
                       pltpu.SemaphoreType.DMA,
                       pltpu.SemaphoreType.DMA,
                       pltpu.SemaphoreType.DMA],
    )
    def k(v_hbm, bt_hbm, bj_hbm, out_hbm, tw0, jw0, tw1, jw1, val0, val1,
          acc_sh, gsem0, gsem1, asem0, asem1, esem0, esem1):
        cid = lax.axis_index("c")
        sid = lax.axis_index("s")
        zrow = jnp.zeros((16,), jnp.float32)
        tws = (tw0, tw1)
        jws = (jw0, jw1)
        vals = (val0, val1)
        esems = (esem0, esem1)
        gsems = (gsem0, gsem1)
        asems = (asem0, asem1)
        nj = win // 128

        def fire_entries(ci, w, par):
            off = ci * ent_chunk + sid * ent_tile + w * win
            for j in range(nj):
                pltpu.async_copy(bt_hbm.at[pl.ds(off + j * 128, 128)],
                                 tws[par].at[j], esems[par])
                pltpu.async_copy(bj_hbm.at[pl.ds(off + j * 128, 128)],
                                 jws[par].at[j], esems[par])

        def drain_entries(par):
            for j in range(nj):
                pltpu.make_async_copy(bt_hbm.at[pl.ds(0, 128)],
                                      tws[par].at[j], esems[par]).wait()
                pltpu.make_async_copy(bj_hbm.at[pl.ds(0, 128)],
                                      jws[par].at[j], esems[par]).wait()

        def fire_gathers(par):
            for j in range(nj):
                pltpu.async_copy(v_hbm.at[tws[par].at[j]],
                                 vals[par].at[pl.ds(j * 128, 128)],
                                 gsems[par])

        def drain_gathers(par):
            for j in range(nj):
                pltpu.make_async_copy(v_hbm.at[pl.ds(0, 128)],
                                      vals[par].at[pl.ds(j * 128, 128)],
                                      gsems[par]).wait()

        def stage(ci, w, par):
            drain_gathers(par)
            adds = [
                pltpu.async_copy(vals[par].at[pl.ds(j * 128, 128)],
                                 acc_sh.at[jws[par].at[j]], asems[par],
                                 add=True)
                for j in range(nj)
            ]

            @pl.when(w + 1 < nwin)
            def _():
                drain_entries(1 - par)
                fire_gathers(1 - par)

            for c in adds:
                c.wait()

            @pl.when(w + 2 < nwin)
            def _():
                fire_entries(ci, w + 2, par)

        def one_pass(p):
            ci = NC * p + cid
            @pl.loop(0, win)
            def _(r):
                @pl.loop(0, w128 // 16)
                def _(q):
                    val0[r, pl.ds(q * 16, 16)] = zrow

            pltpu.sync_copy(val0, acc_sh.at[pl.ds(sid * rpt, win)])
            pltpu.sync_copy(val0.at[pl.ds(0, rpt - win)],
                            acc_sh.at[pl.ds(sid * rpt + win, rpt - win)])
            plsc.subcore_barrier()
            fire_entries(ci, 0, 0)
            fire_entries(ci, 1, 1)
            drain_entries(0)
            fire_gathers(0)

            @pl.loop(0, nwin // 2)
            def _(u):
                stage(ci, u * 2, 0)
                stage(ci, u * 2 + 1, 1)

            plsc.subcore_barrier()
            return ci

        for p in range(npass - 1):
            ci = one_pass(p)
            pltpu.sync_copy(
                acc_sh.at[pl.ds(sid * rpt, rpt)],
                out_hbm.at[pl.ds(ci * CHUNK + sid * rpt, rpt)])

        @pl.when(cid == 0)
        def _():
            ci = one_pass(npass - 1)
            base = (NBIN - 1) * CHUNK

            @pl.when(sid < rem_full)
            def _():
                pltpu.sync_copy(
                    acc_sh.at[pl.ds(sid * rpt, rpt)],
                    out_hbm.at[pl.ds(base + sid * rpt, rpt)])

            @pl.when(sid == rem_full)
            def _():
                pltpu.sync_copy(
                    acc_sh.at[pl.ds(rem_full * rpt, rem_part)],
                    out_hbm.at[pl.ds(base + rem_full * rpt, rem_part)])

    return k(v, bins_t, bins_j)


def kernel(x, rbf, sbf, idx_kj, idx_ji, W_rbf1, W_rbf2, W_sbf1, W_sbf2,
           W_rbf, W_kj, b_kj, W_ji, b_ji, W_down, W_up, W_lin, b_lin,
           res_W1, res_b1, res_W2, res_b2):
    e, h = x.shape
    idx_kj = idx_kj.astype(jnp.int32)
    idx_ji = idx_ji.astype(jnp.int32)
    bf = jnp.bfloat16
    bins_t, bins_j = _sc_bin(idx_ji, idx_ji.shape[0])
    x_ji, xd = _pre(x, rbf,
                    W_ji.astype(bf), b_ji.reshape(1, h),
                    W_kj.astype(bf), b_kj.reshape(1, h),
                    W_rbf1, W_rbf2, W_down.astype(bf))
    g = _sc_gather(xd, idx_kj)
    v = _mid(sbf, g, W_sbf1, W_sbf2)
    seg = _sc_scatter(v, bins_t, bins_j, e)
    e1, e2 = _post(seg, x_ji, x, rbf,
                   W_up.astype(bf),
                   res_W1.astype(bf), res_b1, res_W2.astype(bf), res_b2,
                   W_lin.astype(bf), b_lin.reshape(1, h), W_rbf)
    return (e1, e2)


# speedup vs baseline: 7.9666x; 1.0076x over previous
import dataclasses
import functools

import jax
import jax.numpy as jnp
from jax import lax
from jax.experimental import pallas as pl
from jax.experimental.pallas import tpu as pltpu
from jax.experimental.pallas import tpu_sc as plsc

NC = 2
NS = 16
CHUNK = 7936
NBIN = 21
CAP = 1280
ACC_ROWS = 8064

_SC_CP = pltpu.CompilerParams()
if "needs_layout_passes" in pltpu.CompilerParams.__dataclass_fields__:
    _SC_CP = dataclasses.replace(_SC_CP, needs_layout_passes=False)


def _swish(v):
    return v * (0.5 * jnp.tanh(v * 0.5) + 0.5)


def _swish_bf(v):
    vb = v.astype(jnp.bfloat16)
    half = jnp.bfloat16(0.5)
    return vb * (half * jnp.tanh(vb * half) + half)


def _k1_body(x_ref, rbf_ref, wji, bji, wkj, bkj, wr1, wr2, wdown,
             xji_out, xd_out):
    xb = x_ref[...].astype(jnp.bfloat16)
    xji = jnp.dot(xb, wji[...], preferred_element_type=jnp.float32) + bji[...]
    xji_out[...] = _swish_bf(xji)
    xkj = _swish_bf(jnp.dot(xb, wkj[...], preferred_element_type=jnp.float32)
                    + bkj[...]).astype(jnp.float32)
    rbf_e = jnp.dot(jnp.dot(rbf_ref[...], wr1[...],
                            preferred_element_type=jnp.float32),
                    wr2[...], preferred_element_type=jnp.float32)
    xkj = xkj * rbf_e
    xd = jnp.dot(xkj.astype(jnp.bfloat16), wdown[...],
                 preferred_element_type=jnp.float32)
    xd_out[:, :xd.shape[1]] = _swish(xd)


def _pre(x, rbf, wji, bji, wkj, bkj, wr1, wr2, wdown):
    e, h = x.shape
    nr = rbf.shape[1]
    nint = wdown.shape[1]
    be = 4000
    full = lambda i: (0, 0)
    row = lambda i: (i, 0)
    return pl.pallas_call(
        _k1_body,
        grid=(e // be,),
        in_specs=[
            pl.BlockSpec((be, h), row),
            pl.BlockSpec((be, nr), row),
            pl.BlockSpec((h, h), full),
            pl.BlockSpec((1, h), full),
            pl.BlockSpec((h, h), full),
            pl.BlockSpec((1, h), full),
            pl.BlockSpec((nr, 16), full),
            pl.BlockSpec((16, h), full),
            pl.BlockSpec((h, nint), full),
        ],
        out_specs=[pl.BlockSpec((be, h), row),
                   pl.BlockSpec((be, 2 * nint), row)],
        out_shape=[jax.ShapeDtypeStruct((e, h), jnp.bfloat16),
                   jax.ShapeDtypeStruct((e, 2 * nint), jnp.float32)],
    )(x, rbf, wji, bji, wkj, bkj, wr1, wr2, wdown)


def _k2_body(sbf_ref, g_ref, w1, w2, v_out):
    s16 = jnp.dot(sbf_ref[...], w1[...], preferred_element_type=jnp.float32)
    se = jnp.dot(s16, w2[...], preferred_element_type=jnp.float32)
    prod = se * g_ref[:, :se.shape[1]]
    v_out[:, :prod.shape[1]] = prod


def _mid(sbf, g, w1, w2):
    t, nsb = sbf.shape
    nint = w2.shape[1]
    bt = 16000
    full = lambda i: (0, 0)
    row = lambda i: (i, 0)
    return pl.pallas_call(
        _k2_body,
        grid=(t // bt,),
        in_specs=[
            pl.BlockSpec((bt, nsb), row),
            pl.BlockSpec((bt, 2 * nint), row),
            pl.BlockSpec((nsb, 16), full),
            pl.BlockSpec((16, nint), full),
        ],
        out_specs=[pl.BlockSpec((bt, 2 * nint), row)],
        out_shape=[jax.ShapeDtypeStruct((t, 2 * nint), jnp.float32)],
    )(sbf, g, w1, w2)[0]


def _k3_body(seg_ref, xji_ref, x_ref, rbf_ref, wup,
             rw10, rw11, rw12, rw20, rw21, rw22,
             rb10, rb11, rb12, rb20, rb21, rb22,
             wlin, blin, wrbf, e1_out, e2_out):
    def res_layer(h, w1, b1, w2, b2):
        u = _swish_bf(jnp.dot(h.astype(jnp.bfloat16), w1[...],
                              preferred_element_type=jnp.float32) + b1[...])
        u = _swish_bf(jnp.dot(u, w2[...],
                              preferred_element_type=jnp.float32) + b2[...])
        return h + u.astype(jnp.float32)

    nint = wup.shape[0]
    xkj = _swish_bf(jnp.dot(seg_ref[:, :nint].astype(jnp.bfloat16), wup[...],
                            preferred_element_type=jnp.float32))
    e1 = xji_ref[...].astype(jnp.float32) + xkj.astype(jnp.float32)
    e1 = res_layer(e1, rw10, rb10, rw20, rb20)
    e1 = _swish_bf(jnp.dot(e1.astype(jnp.bfloat16), wlin[...],
                           preferred_element_type=jnp.float32) + blin[...]
                   ).astype(jnp.float32)
    e1 = e1 + x_ref[...]
    e1 = res_layer(e1, rw11, rb11, rw21, rb21)
    e1 = res_layer(e1, rw12, rb12, rw22, rb22)
    e1_out[...] = e1
    e2_out[...] = jnp.dot(rbf_ref[...], wrbf[...],
                          preferred_element_type=jnp.float32) * e1


def _post(seg, x_ji, x, rbf, wup, rws1, rbs1, rws2, rbs2, wlin, blin, wrbf):
    e, h = x.shape
    nr = rbf.shape[1]
    nint = wup.shape[0]
    be = 4000
    full = lambda i: (0, 0)
    row = lambda i: (i, 0)
    rw1 = [rws1[i] for i in range(3)]
    rw2 = [rws2[i] for i in range(3)]
    rb1 = [rbs1[i:i + 1] for i in range(3)]
    rb2 = [rbs2[i:i + 1] for i in range(3)]
    wspec = [pl.BlockSpec((h, h), full)] * 6
    bspec = [pl.BlockSpec((1, h), full)] * 6
    return pl.pallas_call(
        _k3_body,
        grid=(e // be,),
        in_specs=[
            pl.BlockSpec((be, 2 * nint), row),
            pl.BlockSpec((be, h), row),
            pl.BlockSpec((be, h), row),
            pl.BlockSpec((be, nr), row),
            pl.BlockSpec((nint, h), full),
        ] + wspec + bspec + [
            pl.BlockSpec((h, h), full),
            pl.BlockSpec((1, h), full),
            pl.BlockSpec((nr, h), full),
        ],
        out_specs=[pl.BlockSpec((be, h), row), pl.BlockSpec((be, h), row)],
        out_shape=[jax.ShapeDtypeStruct((e, h), jnp.float32),
                   jax.ShapeDtypeStruct((e, h), jnp.float32)],
    )(seg, x_ji, x, rbf, wup, *rw1, *rw2, *rb1, *rb2, wlin, blin, wrbf)


def _sc_gather(xd, idx):
    e, w128 = xd.shape
    t = idx.shape[0]
    gw = 256
    nstr = gw // 128
    nwin = t // gw
    nw = NC * NS
    outer = (nwin + nw - 1) // nw
    mesh = plsc.VectorSubcoreMesh(core_axis_name="c", subcore_axis_name="s")

    @functools.partial(
        pl.kernel, mesh=mesh,
        out_type=jax.ShapeDtypeStruct((t, w128), jnp.float32),
        scratch_types=[pltpu.VMEM((nstr, 128), jnp.int32),
                       pltpu.VMEM((nstr, 128), jnp.int32),
                       pltpu.VMEM((gw, w128), jnp.float32),
                       pltpu.VMEM((gw, w128), jnp.float32),
                       pltpu.SemaphoreType.DMA,
                       pltpu.SemaphoreType.DMA,
                       pltpu.SemaphoreType.DMA,
                       pltpu.SemaphoreType.DMA,
                       pltpu.SemaphoreType.DMA,
                       pltpu.SemaphoreType.DMA],
    )
    def k(xd_hbm, idx_hbm, g_hbm, idx0, idx1, rows0, rows1,
          gsem0, gsem1, isem0, isem1, wsem0, wsem1):
        cid = lax.axis_index("c")
        sid = lax.axis_index("s")
        wid = sid * NC + cid
        ibufs = (idx0, idx1)
        rbufs = (rows0, rows1)
        isems = (isem0, isem1)
        gsems = (gsem0, gsem1)
        wsems = (wsem0, wsem1)

        def fire_idx(wi, par):
            @pl.when(wi < nwin)
            def _():
                t0 = wi * gw
                for k8 in range(nstr):
                    pltpu.async_copy(idx_hbm.at[pl.ds(t0 + k8 * 128, 128)],
                                     ibufs[par].at[k8], isems[par])

        def do_window(wi, par):
            @pl.when(wi < nwin)
            def _():
                @pl.when(wi >= 2 * nw)
                def _():
                    pltpu.make_async_copy(
                        rbufs[par], g_hbm.at[pl.ds(0, gw)], wsems[par]).wait()

                for k8 in range(nstr):
                    pltpu.make_async_copy(idx_hbm.at[pl.ds(0, 128)],
                                          ibufs[par].at[k8],
                                          isems[par]).wait()
                fire_idx(wi + nw, 1 - par)
                copies = [
                    pltpu.async_copy(xd_hbm.at[ibufs[par].at[k8]],
                                     rbufs[par].at[pl.ds(k8 * 128, 128)],
                                     gsems[par])
                    for k8 in range(nstr)
                ]
                for c in copies:
                    c.wait()
                pltpu.async_copy(rbufs[par], g_hbm.at[pl.ds(wi * gw, gw)],
                                 wsems[par])

        fire_idx(wid, 0)

        @pl.loop(0, (outer + 1) // 2)
        def _(u):
            do_window(u * 2 * nw + wid, 0)
            do_window((u * 2 + 1) * nw + wid, 1)

        for par in range(2):
            pltpu.make_async_copy(rbufs[par], g_hbm.at[pl.ds(0, gw)],
                                  wsems[par]).wait()

    return k(xd, idx)


def _sc_bin(idx, t):
    nw = NC * NS
    win = 256
    nwin = t // win
    outer = (nwin + nw - 1) // nw
    mesh = plsc.VectorSubcoreMesh(core_axis_name="c", subcore_axis_name="s")
    nent = NBIN * nw * CAP

    @functools.partial(
        pl.kernel, mesh=mesh, compiler_params=_SC_CP,
        out_type=[jax.ShapeDtypeStruct((nent,), jnp.int32),
                  jax.ShapeDtypeStruct((nent,), jnp.int32)],
        scratch_types=[pltpu.VMEM((win,), jnp.int32),
                       pltpu.VMEM((NBIN * CAP,), jnp.int32),
                       pltpu.VMEM((NBIN * CAP,), jnp.int32),
                       pltpu.SMEM((NBIN,), jnp.int32),
                       pltpu.SemaphoreType.DMA],
    )
    def k(ji_hbm, bt_hbm, bj_hbm, jiw, tbuf, jbuf, ptrs, sem):
        cid = lax.axis_index("c")
        sid = lax.axis_index("s")
        wid = sid * NC + cid
        lanes = lax.iota(jnp.int32, 16)
        padt = (wid * 16 + lanes) * 64
        padj = CHUNK + wid * 4 + (lanes & 3)

        @pl.loop(0, NBIN * CAP // 16)
        def _(z):
            tbuf[pl.ds(z * 16, 16)] = padt
            jbuf[pl.ds(z * 16, 16)] = padj

        for b in range(NBIN):
            ptrs[b] = 0

        @pl.loop(0, outer)
        def _(w):
            wi = w * nw + wid

            @pl.when(wi < nwin)
            def _():
                t0 = wi * win
                pltpu.sync_copy(ji_hbm.at[pl.ds(t0, win)], jiw)

                @pl.loop(0, win // 16)
                def _(i):
                    ji16 = jiw[pl.ds(i * 16, 16)]
                    binv = ji16 // CHUNK
                    tvec = t0 + i * 16 + lanes
                    for b in range(NBIN):
                        m = binv == b
                        mi = m.astype(jnp.int32)
                        cnt = jnp.sum(mi)

                        @pl.when(cnt > 0)
                        def _():
                            cum = plsc.cumsum(mi)
                            ptr = ptrs[b]
                            pos = b * CAP + ptr + cum - 1
                            plsc.store_scatter(tbuf, [pos], tvec, mask=m)
                            plsc.store_scatter(jbuf, [pos], ji16 - b * CHUNK,
                                               mask=m)
                            ptrs[b] = ptr + cnt

        for b in range(NBIN):
            dst = (b * nw + wid) * CAP
            pltpu.sync_copy(tbuf.at[pl.ds(b * CAP, CAP)],
                            bt_hbm.at[pl.ds(dst, CAP)])
            pltpu.sync_copy(jbuf.at[pl.ds(b * CAP, CAP)],
                            bj_hbm.at[pl.ds(dst, CAP)])

    return k(idx)


def _sc_scatter(v, bins_t, bins_j, e):
    t, w128 = v.shape
    nw = NC * NS
    rpt = CHUNK // NS
    ent_chunk = nw * CAP
    ent_tile = ent_chunk // NS
    win = 256
    nwin = ent_tile // win
    npass = (NBIN + 1) // NC
    rem = e - (NBIN - 1) * CHUNK
    rem_full = rem // rpt
    rem_part = rem - rem_full * rpt
    mesh = plsc.VectorSubcoreMesh(core_axis_name="c", subcore_axis_name="s")

    @functools.partial(
        pl.kernel, mesh=mesh,
        out_type=jax.ShapeDtypeStruct((e, w128), jnp.float32),
        scratch_types=[pltpu.VMEM((2, 128), jnp.int32),
                       pltpu.VMEM((2, 128), jnp.int32),
                       pltpu.VMEM((2, 128), jnp.int32),
                       pltpu.VMEM((2, 128), jnp.int32),
                       pltpu.VMEM((win, w128), jnp.float32),
                       pltpu.VMEM((win, w128), jnp.float32),
                       pltpu.VMEM_SHARED((ACC_ROWS, w128), jnp.float32),
                       pltpu.SemaphoreType.DMA,
                       pltpu.SemaphoreType.DMA,
                       pltpu.SemaphoreType.DMA,
                       pltpu.SemaphoreType.DMA,
                       pltpu.SemaphoreType.DMA,
                       pltpu.SemaphoreType.DMA],
    )
    def k(v_hbm, bt_hbm, bj_hbm, out_hbm, tw0, jw0, tw1, jw1, val0, val1,
          acc_sh, gsem0, gsem1, asem0, asem1, esem0, esem1):
        cid = lax.axis_index("c")
        sid = lax.axis_index("s")
        zrow = jnp.zeros((16,), jnp.float32)
        tws = (tw0, tw1)
        jws = (jw0, jw1)
        vals = (val0, val1)
        esems = (esem0, esem1)
        gsems = (gsem0, gsem1)
        asems = (asem0, asem1)
        nj = win // 128

        def fire_entries(ci, w, par):
            off = ci * ent_chunk + sid * ent_tile + w * win
            for j in range(nj):
                pltpu.async_copy(bt_hbm.at[pl.ds(off + j * 128, 128)],
                                 tws[par].at[j], esems[par])
                pltpu.async_copy(bj_hbm.at[pl.ds(off + j * 128, 128)],
                                 jws[par].at[j], esems[par])

        def drain_entries(par):
            for j in range(nj):
                pltpu.make_async_copy(bt_hbm.at[pl.ds(0, 128)],
                                      tws[par].at[j], esems[par]).wait()
                pltpu.make_async_copy(bj_hbm.at[pl.ds(0, 128)],
                                      jws[par].at[j], esems[par]).wait()

        def fire_gathers(par):
            for j in range(nj):
                pltpu.async_copy(v_hbm.at[tws[par].at[j]],
                                 vals[par].at[pl.ds(j * 128, 128)],
                                 gsems[par])

        def drain_gathers(par):
            for j in range(nj):
                pltpu.make_async_copy(v_hbm.at[pl.ds(0, 128)],
                                      vals[par].at[pl.ds(j * 128, 128)],
                                      gsems[par]).wait()

        def stage(ci, w, par):
            drain_gathers(par)
            adds = [
                pltpu.async_copy(vals[par].at[pl.ds(j * 128, 128)],
                                 acc_sh.at[jws[par].at[j]], asems[par],
                                 add=True)
                for j in range(nj)
            ]

            @pl.when(w + 1 < nwin)
            def _():
                drain_entries(1 - par)
                fire_gathers(1 - par)

            for c in adds:
                c.wait()

            @pl.when(w + 2 < nwin)
            def _():
                fire_entries(ci, w + 2, par)

        def one_pass(p):
            ci = NC * p + cid
            @pl.loop(0, win)
            def _(r):
                @pl.loop(0, w128 // 16)
                def _(q):
                    val0[r, pl.ds(q * 16, 16)] = zrow

            pltpu.sync_copy(val0, acc_sh.at[pl.ds(sid * rpt, win)])
            pltpu.sync_copy(val0.at[pl.ds(0, rpt - win)],
                            acc_sh.at[pl.ds(sid * rpt + win, rpt - win)])
            plsc.subcore_barrier()
            fire_entries(ci, 0, 0)
            fire_entries(ci, 1, 1)
            drain_entries(0)
            fire_gathers(0)

            @pl.loop(0, nwin // 2)
            def _(u):
                stage(ci, u * 2, 0)
                stage(ci, u * 2 + 1, 1)

            plsc.subcore_barrier()
            return ci

        for p in range(npass - 1):
            ci = one_pass(p)
            pltpu.sync_copy(
                acc_sh.at[pl.ds(sid * rpt, rpt)],
                out_hbm.at[pl.ds(ci * CHUNK + sid * rpt, rpt)])

        @pl.when(cid == 0)
        def _():
            ci = one_pass(npass - 1)
            base = (NBIN - 1) * CHUNK

            @pl.when(sid < rem_full)
            def _():
                pltpu.sync_copy(
                    acc_sh.at[pl.ds(sid * rpt, rpt)],
                    out_hbm.at[pl.ds(base + sid * rpt, rpt)])

            @pl.when(sid == rem_full)
            def _():
                pltpu.sync_copy(
                    acc_sh.at[pl.ds(rem_full * rpt, rem_part)],
                    out_hbm.at[pl.ds(base + rem_full * rpt, rem_part)])

    return k(v, bins_t, bins_j)


def kernel(x, rbf, sbf, idx_kj, idx_ji, W_rbf1, W_rbf2, W_sbf1, W_sbf2,
           W_rbf, W_kj, b_kj, W_ji, b_ji, W_down, W_up, W_lin, b_lin,
           res_W1, res_b1, res_W2, res_b2):
    e, h = x.shape
    idx_kj = idx_kj.astype(jnp.int32)
    idx_ji = idx_ji.astype(jnp.int32)
    bf = jnp.bfloat16
    bins_t, bins_j = _sc_bin(idx_ji, idx_ji.shape[0])
    x_ji, xd = _pre(x, rbf,
                    W_ji.astype(bf), b_ji.reshape(1, h),
                    W_kj.astype(bf), b_kj.reshape(1, h),
                    W_rbf1, W_rbf2, W_down.astype(bf))
    g = _sc_gather(xd, idx_kj)
    v = _mid(sbf, g, W_sbf1, W_sbf2)
    seg = _sc_scatter(v, bins_t, bins_j, e)
    e1, e2 = _post(seg, x_ji, x, rbf,
                   W_up.astype(bf),
                   res_W1.astype(bf), res_b1, res_W2.astype(bf), res_b2,
                   W_lin.astype(bf), b_lin.reshape(1, h), W_rbf)
    return (e1, e2)


# speedup vs baseline: 8.0108x; 1.0056x over previous
import dataclasses
import functools

import jax
import jax.numpy as jnp
from jax import lax
from jax.experimental import pallas as pl
from jax.experimental.pallas import tpu as pltpu
from jax.experimental.pallas import tpu_sc as plsc

NC = 2
NS = 16
CHUNK = 7936
NBIN = 21
CAP = 1280
ACC_ROWS = 8064

_SC_CP = pltpu.CompilerParams()
if "needs_layout_passes" in pltpu.CompilerParams.__dataclass_fields__:
    _SC_CP = dataclasses.replace(_SC_CP, needs_layout_passes=False)


def _swish(v):
    return v * (0.5 * jnp.tanh(v * 0.5) + 0.5)


def _swish_bf(v):
    vb = v.astype(jnp.bfloat16)
    half = jnp.bfloat16(0.5)
    return vb * (half * jnp.tanh(vb * half) + half)


def _k1_body(x_ref, rbf_ref, wji, bji, wkj, bkj, wr1, wr2, wdown,
             xji_out, xd_out):
    xb = x_ref[...].astype(jnp.bfloat16)
    xji = jnp.dot(xb, wji[...], preferred_element_type=jnp.float32) + bji[...]
    xji_out[...] = _swish_bf(xji)
    xkj = _swish_bf(jnp.dot(xb, wkj[...], preferred_element_type=jnp.float32)
                    + bkj[...]).astype(jnp.float32)
    rbf_e = jnp.dot(jnp.dot(rbf_ref[...], wr1[...],
                            preferred_element_type=jnp.float32),
                    wr2[...], preferred_element_type=jnp.float32)
    xkj = xkj * rbf_e
    xd = jnp.dot(xkj.astype(jnp.bfloat16), wdown[...],
                 preferred_element_type=jnp.float32)
    xd_out[:, :xd.shape[1]] = _swish(xd)


def _pre(x, rbf, wji, bji, wkj, bkj, wr1, wr2, wdown):
    e, h = x.shape
    nr = rbf.shape[1]
    nint = wdown.shape[1]
    be = 8000
    full = lambda i: (0, 0)
    row = lambda i: (i, 0)
    return pl.pallas_call(
        _k1_body,
        grid=(e // be,),
        in_specs=[
            pl.BlockSpec((be, h), row),
            pl.BlockSpec((be, nr), row),
            pl.BlockSpec((h, h), full),
            pl.BlockSpec((1, h), full),
            pl.BlockSpec((h, h), full),
            pl.BlockSpec((1, h), full),
            pl.BlockSpec((nr, 16), full),
            pl.BlockSpec((16, h), full),
            pl.BlockSpec((h, nint), full),
        ],
        out_specs=[pl.BlockSpec((be, h), row),
                   pl.BlockSpec((be, 2 * nint), row)],
        out_shape=[jax.ShapeDtypeStruct((e, h), jnp.bfloat16),
                   jax.ShapeDtypeStruct((e, 2 * nint), jnp.float32)],
    )(x, rbf, wji, bji, wkj, bkj, wr1, wr2, wdown)


def _k2_body(sbf_ref, g_ref, w1, w2, v_out):
    s16 = jnp.dot(sbf_ref[...], w1[...], preferred_element_type=jnp.float32)
    se = jnp.dot(s16, w2[...], preferred_element_type=jnp.float32)
    prod = se * g_ref[:, :se.shape[1]]
    v_out[:, :prod.shape[1]] = prod


def _mid(sbf, g, w1, w2):
    t, nsb = sbf.shape
    nint = w2.shape[1]
    bt = 16000
    full = lambda i: (0, 0)
    row = lambda i: (i, 0)
    return pl.pallas_call(
        _k2_body,
        grid=(t // bt,),
        in_specs=[
            pl.BlockSpec((bt, nsb), row),
            pl.BlockSpec((bt, 2 * nint), row),
            pl.BlockSpec((nsb, 16), full),
            pl.BlockSpec((16, nint), full),
        ],
        out_specs=[pl.BlockSpec((bt, 2 * nint), row)],
        out_shape=[jax.ShapeDtypeStruct((t, 2 * nint), jnp.float32)],
    )(sbf, g, w1, w2)[0]


def _k3_body(seg_ref, xji_ref, x_ref, rbf_ref, wup,
             rw10, rw11, rw12, rw20, rw21, rw22,
             rb10, rb11, rb12, rb20, rb21, rb22,
             wlin, blin, wrbf, e1_out, e2_out):
    def res_layer(h, w1, b1, w2, b2):
        u = _swish_bf(jnp.dot(h.astype(jnp.bfloat16), w1[...],
                              preferred_element_type=jnp.float32) + b1[...])
        u = _swish_bf(jnp.dot(u, w2[...],
                              preferred_element_type=jnp.float32) + b2[...])
        return h + u.astype(jnp.float32)

    nint = wup.shape[0]
    xkj = _swish_bf(jnp.dot(seg_ref[:, :nint].astype(jnp.bfloat16), wup[...],
                            preferred_element_type=jnp.float32))
    e1 = xji_ref[...].astype(jnp.float32) + xkj.astype(jnp.float32)
    e1 = res_layer(e1, rw10, rb10, rw20, rb20)
    e1 = _swish_bf(jnp.dot(e1.astype(jnp.bfloat16), wlin[...],
                           preferred_element_type=jnp.float32) + blin[...]
                   ).astype(jnp.float32)
    e1 = e1 + x_ref[...]
    e1 = res_layer(e1, rw11, rb11, rw21, rb21)
    e1 = res_layer(e1, rw12, rb12, rw22, rb22)
    e1_out[...] = e1
    e2_out[...] = jnp.dot(rbf_ref[...], wrbf[...],
                          preferred_element_type=jnp.float32) * e1


def _post(seg, x_ji, x, rbf, wup, rws1, rbs1, rws2, rbs2, wlin, blin, wrbf):
    e, h = x.shape
    nr = rbf.shape[1]
    nint = wup.shape[0]
    be = 4000
    full = lambda i: (0, 0)
    row = lambda i: (i, 0)
    rw1 = [rws1[i] for i in range(3)]
    rw2 = [rws2[i] for i in range(3)]
    rb1 = [rbs1[i:i + 1] for i in range(3)]
    rb2 = [rbs2[i:i + 1] for i in range(3)]
    wspec = [pl.BlockSpec((h, h), full)] * 6
    bspec = [pl.BlockSpec((1, h), full)] * 6
    return pl.pallas_call(
        _k3_body,
        grid=(e // be,),
        in_specs=[
            pl.BlockSpec((be, 2 * nint), row),
            pl.BlockSpec((be, h), row),
            pl.BlockSpec((be, h), row),
            pl.BlockSpec((be, nr), row),
            pl.BlockSpec((nint, h), full),
        ] + wspec + bspec + [
            pl.BlockSpec((h, h), full),
            pl.BlockSpec((1, h), full),
            pl.BlockSpec((nr, h), full),
        ],
        out_specs=[pl.BlockSpec((be, h), row), pl.BlockSpec((be, h), row)],
        out_shape=[jax.ShapeDtypeStruct((e, h), jnp.float32),
                   jax.ShapeDtypeStruct((e, h), jnp.float32)],
    )(seg, x_ji, x, rbf, wup, *rw1, *rw2, *rb1, *rb2, wlin, blin, wrbf)


def _sc_gather(xd, idx):
    e, w128 = xd.shape
    t = idx.shape[0]
    gw = 256
    nstr = gw // 128
    nwin = t // gw
    nw = NC * NS
    outer = (nwin + nw - 1) // nw
    mesh = plsc.VectorSubcoreMesh(core_axis_name="c", subcore_axis_name="s")

    @functools.partial(
        pl.kernel, mesh=mesh,
        out_type=jax.ShapeDtypeStruct((t, w128), jnp.float32),
        scratch_types=[pltpu.VMEM((nstr, 128), jnp.int32),
                       pltpu.VMEM((nstr, 128), jnp.int32),
                       pltpu.VMEM((gw, w128), jnp.float32),
                       pltpu.VMEM((gw, w128), jnp.float32),
                       pltpu.SemaphoreType.DMA,
                       pltpu.SemaphoreType.DMA,
                       pltpu.SemaphoreType.DMA,
                       pltpu.SemaphoreType.DMA,
                       pltpu.SemaphoreType.DMA,
                       pltpu.SemaphoreType.DMA],
    )
    def k(xd_hbm, idx_hbm, g_hbm, idx0, idx1, rows0, rows1,
          gsem0, gsem1, isem0, isem1, wsem0, wsem1):
        cid = lax.axis_index("c")
        sid = lax.axis_index("s")
        wid = sid * NC + cid
        ibufs = (idx0, idx1)
        rbufs = (rows0, rows1)
        isems = (isem0, isem1)
        gsems = (gsem0, gsem1)
        wsems = (wsem0, wsem1)

        def fire_idx(wi, par):
            @pl.when(wi < nwin)
            def _():
                t0 = wi * gw
                for k8 in range(nstr):
                    pltpu.async_copy(idx_hbm.at[pl.ds(t0 + k8 * 128, 128)],
                                     ibufs[par].at[k8], isems[par])

        def do_window(wi, par):
            @pl.when(wi < nwin)
            def _():
                @pl.when(wi >= 2 * nw)
                def _():
                    pltpu.make_async_copy(
                        rbufs[par], g_hbm.at[pl.ds(0, gw)], wsems[par]).wait()

                for k8 in range(nstr):
                    pltpu.make_async_copy(idx_hbm.at[pl.ds(0, 128)],
                                          ibufs[par].at[k8],
                                          isems[par]).wait()
                fire_idx(wi + nw, 1 - par)
                copies = [
                    pltpu.async_copy(xd_hbm.at[ibufs[par].at[k8]],
                                     rbufs[par].at[pl.ds(k8 * 128, 128)],
                                     gsems[par])
                    for k8 in range(nstr)
                ]
                for c in copies:
                    c.wait()
                pltpu.async_copy(rbufs[par], g_hbm.at[pl.ds(wi * gw, gw)],
                                 wsems[par])

        fire_idx(wid, 0)

        @pl.loop(0, (outer + 1) // 2)
        def _(u):
            do_window(u * 2 * nw + wid, 0)
            do_window((u * 2 + 1) * nw + wid, 1)

        for par in range(2):
            pltpu.make_async_copy(rbufs[par], g_hbm.at[pl.ds(0, gw)],
                                  wsems[par]).wait()

    return k(xd, idx)


def _sc_bin(idx, t):
    nw = NC * NS
    win = 256
    nwin = t // win
    outer = (nwin + nw - 1) // nw
    mesh = plsc.VectorSubcoreMesh(core_axis_name="c", subcore_axis_name="s")
    nent = NBIN * nw * CAP

    @functools.partial(
        pl.kernel, mesh=mesh, compiler_params=_SC_CP,
        out_type=[jax.ShapeDtypeStruct((nent,), jnp.int32),
                  jax.ShapeDtypeStruct((nent,), jnp.int32)],
        scratch_types=[pltpu.VMEM((win,), jnp.int32),
                       pltpu.VMEM((NBIN * CAP,), jnp.int32),
                       pltpu.VMEM((NBIN * CAP,), jnp.int32),
                       pltpu.SMEM((NBIN,), jnp.int32),
                       pltpu.SemaphoreType.DMA],
    )
    def k(ji_hbm, bt_hbm, bj_hbm, jiw, tbuf, jbuf, ptrs, sem):
        cid = lax.axis_index("c")
        sid = lax.axis_index("s")
        wid = sid * NC + cid
        lanes = lax.iota(jnp.int32, 16)
        padt = (wid * 16 + lanes) * 64
        padj = CHUNK + wid * 4 + (lanes & 3)

        @pl.loop(0, NBIN * CAP // 16)
        def _(z):
            tbuf[pl.ds(z * 16, 16)] = padt
            jbuf[pl.ds(z * 16, 16)] = padj

        for b in range(NBIN):
            ptrs[b] = 0

        @pl.loop(0, outer)
        def _(w):
            wi = w * nw + wid

            @pl.when(wi < nwin)
            def _():
                t0 = wi * win
                pltpu.sync_copy(ji_hbm.at[pl.ds(t0, win)], jiw)

                @pl.loop(0, win // 16)
                def _(i):
                    ji16 = jiw[pl.ds(i * 16, 16)]
                    binv = ji16 // CHUNK
                    tvec = t0 + i * 16 + lanes
                    for b in range(NBIN):
                        m = binv == b
                        mi = m.astype(jnp.int32)
                        cnt = jnp.sum(mi)

                        @pl.when(cnt > 0)
                        def _():
                            cum = plsc.cumsum(mi)
                            ptr = ptrs[b]
                            pos = b * CAP + ptr + cum - 1
                            plsc.store_scatter(tbuf, [pos], tvec, mask=m)
                            plsc.store_scatter(jbuf, [pos], ji16 - b * CHUNK,
                                               mask=m)
                            ptrs[b] = ptr + cnt

        for b in range(NBIN):
            dst = (b * nw + wid) * CAP
            pltpu.sync_copy(tbuf.at[pl.ds(b * CAP, CAP)],
                            bt_hbm.at[pl.ds(dst, CAP)])
            pltpu.sync_copy(jbuf.at[pl.ds(b * CAP, CAP)],
                            bj_hbm.at[pl.ds(dst, CAP)])

    return k(idx)


def _sc_scatter(v, bins_t, bins_j, e):
    t, w128 = v.shape
    nw = NC * NS
    rpt = CHUNK // NS
    ent_chunk = nw * CAP
    ent_tile = ent_chunk // NS
    win = 256
    nwin = ent_tile // win
    npass = (NBIN + 1) // NC
    rem = e - (NBIN - 1) * CHUNK
    rem_full = rem // rpt
    rem_part = rem - rem_full * rpt
    mesh = plsc.VectorSubcoreMesh(core_axis_name="c", subcore_axis_name="s")

    @functools.partial(
        pl.kernel, mesh=mesh,
        out_type=jax.ShapeDtypeStruct((e, w128), jnp.float32),
        scratch_types=[pltpu.VMEM((2, 128), jnp.int32),
                       pltpu.VMEM((2, 128), jnp.int32),
                       pltpu.VMEM((2, 128), jnp.int32),
                       pltpu.VMEM((2, 128), jnp.int32),
                       pltpu.VMEM((win, w128), jnp.float32),
                       pltpu.VMEM((win, w128), jnp.float32),
                       pltpu.VMEM_SHARED((ACC_ROWS, w128), jnp.float32),
                       pltpu.SemaphoreType.DMA,
                       pltpu.SemaphoreType.DMA,
                       pltpu.SemaphoreType.DMA,
                       pltpu.SemaphoreType.DMA,
                       pltpu.SemaphoreType.DMA,
                       pltpu.SemaphoreType.DMA],
    )
    def k(v_hbm, bt_hbm, bj_hbm, out_hbm, tw0, jw0, tw1, jw1, val0, val1,
          acc_sh, gsem0, gsem1, asem0, asem1, esem0, esem1):
        cid = lax.axis_index("c")
        sid = lax.axis_index("s")
        zrow = jnp.zeros((16,), jnp.float32)
        tws = (tw0, tw1)
        jws = (jw0, jw1)
        vals = (val0, val1)
        esems = (esem0, esem1)
        gsems = (gsem0, gsem1)
        asems = (asem0, asem1)
        nj = win // 128

        def fire_entries(ci, w, par):
            off = ci * ent_chunk + sid * ent_tile + w * win
            for j in range(nj):
                pltpu.async_copy(bt_hbm.at[pl.ds(off + j * 128, 128)],
                                 tws[par].at[j], esems[par])
                pltpu.async_copy(bj_hbm.at[pl.ds(off + j * 128, 128)],
                                 jws[par].at[j], esems[par])

        def drain_entries(par):
            for j in range(nj):
                pltpu.make_async_copy(bt_hbm.at[pl.ds(0, 128)],
                                      tws[par].at[j], esems[par]).wait()
                pltpu.make_async_copy(bj_hbm.at[pl.ds(0, 128)],
                                      jws[par].at[j], esems[par]).wait()

        def fire_gathers(par):
            for j in range(nj):
                pltpu.async_copy(v_hbm.at[tws[par].at[j]],
                                 vals[par].at[pl.ds(j * 128, 128)],
                                 gsems[par])

        def drain_gathers(par):
            for j in range(nj):
                pltpu.make_async_copy(v_hbm.at[pl.ds(0, 128)],
                                      vals[par].at[pl.ds(j * 128, 128)],
                                      gsems[par]).wait()

        def stage(ci, w, par):
            drain_gathers(par)
            adds = [
                pltpu.async_copy(vals[par].at[pl.ds(j * 128, 128)],
                                 acc_sh.at[jws[par].at[j]], asems[par],
                                 add=True)
                for j in range(nj)
            ]

            @pl.when(w + 1 < nwin)
            def _():
                drain_entries(1 - par)
                fire_gathers(1 - par)

            for c in adds:
                c.wait()

            @pl.when(w + 2 < nwin)
            def _():
                fire_entries(ci, w + 2, par)

        def one_pass(p):
            ci = NC * p + cid
            @pl.loop(0, win)
            def _(r):
                @pl.loop(0, w128 // 16)
                def _(q):
                    val0[r, pl.ds(q * 16, 16)] = zrow

            pltpu.sync_copy(val0, acc_sh.at[pl.ds(sid * rpt, win)])
            pltpu.sync_copy(val0.at[pl.ds(0, rpt - win)],
                            acc_sh.at[pl.ds(sid * rpt + win, rpt - win)])
            plsc.subcore_barrier()
            fire_entries(ci, 0, 0)
            fire_entries(ci, 1, 1)
            drain_entries(0)
            fire_gathers(0)

            @pl.loop(0, nwin // 2)
            def _(u):
                stage(ci, u * 2, 0)
                stage(ci, u * 2 + 1, 1)

            plsc.subcore_barrier()
            return ci

        for p in range(npass - 1):
            ci = one_pass(p)
            pltpu.sync_copy(
                acc_sh.at[pl.ds(sid * rpt, rpt)],
                out_hbm.at[pl.ds(ci * CHUNK + sid * rpt, rpt)])

        @pl.when(cid == 0)
        def _():
            ci = one_pass(npass - 1)
            base = (NBIN - 1) * CHUNK

            @pl.when(sid < rem_full)
            def _():
                pltpu.sync_copy(
                    acc_sh.at[pl.ds(sid * rpt, rpt)],
                    out_hbm.at[pl.ds(base + sid * rpt, rpt)])

            @pl.when(sid == rem_full)
            def _():
                pltpu.sync_copy(
                    acc_sh.at[pl.ds(rem_full * rpt, rem_part)],
                    out_hbm.at[pl.ds(base + rem_full * rpt, rem_part)])

    return k(v, bins_t, bins_j)


def kernel(x, rbf, sbf, idx_kj, idx_ji, W_rbf1, W_rbf2, W_sbf1, W_sbf2,
           W_rbf, W_kj, b_kj, W_ji, b_ji, W_down, W_up, W_lin, b_lin,
           res_W1, res_b1, res_W2, res_b2):
    e, h = x.shape
    idx_kj = idx_kj.astype(jnp.int32)
    idx_ji = idx_ji.astype(jnp.int32)
    bf = jnp.bfloat16
    bins_t, bins_j = _sc_bin(idx_ji, idx_ji.shape[0])
    x_ji, xd = _pre(x, rbf,
                    W_ji.astype(bf), b_ji.reshape(1, h),
                    W_kj.astype(bf), b_kj.reshape(1, h),
                    W_rbf1, W_rbf2, W_down.astype(bf))
    g = _sc_gather(xd, idx_kj)
    v = _mid(sbf, g, W_sbf1, W_sbf2)
    seg = _sc_scatter(v, bins_t, bins_j, e)
    e1, e2 = _post(seg, x_ji, x, rbf,
                   W_up.astype(bf),
                   res_W1.astype(bf), res_b1, res_W2.astype(bf), res_b2,
                   W_lin.astype(bf), b_lin.reshape(1, h), W_rbf)
    return (e1, e2)
